# Initial kernel scaffold; baseline (speedup 1.0000x reference)
import jax
import jax.numpy as jnp
from jax.experimental import pallas as pl


def kernel(node_input, node_attr, edge_src, edge_dst, edge_attr, edge_scalars, num_neighbors, W_sc, W_lin1, W_fc1, W_fc2, W_lin2, W_lin3):
    raise NotImplementedError("write your pallas kernel here")


# speedup vs baseline: 1.7978x; 1.7978x over previous
import functools
import math

import jax
import jax.numpy as jnp
from jax import lax
from jax.experimental import pallas as pl
from jax.experimental.pallas import tpu as pltpu
from jax.experimental.pallas import tpu_sc as plsc

_N = 10000
_E = 320000
_D = 128
_A = 8
_S = 16
_H = 64

_NC = 2
_NS = 16
_NW = _NC * _NS
_EPW = _E // _NW
_C = 80
_NCHUNK = _EPW // _C
_NP = 10240
_RPS = _NP // _NS
_RC = 128

_INV_FAN = 1.0 / math.sqrt(float(_D * _A))


def _node_prep_body(x_ref, a_ref, wsc_ref, wl1_ref, nsc_ref, nf_ref):
    x = x_ref[...]
    a = a_ref[...]
    acc_sc = jnp.zeros(x.shape, jnp.float32)
    acc_l1 = jnp.zeros(x.shape, jnp.float32)
    for j in range(_A):
        aj = a[:, j:j + 1]
        acc_sc += aj * jnp.dot(x, wsc_ref[j], preferred_element_type=jnp.float32)
        acc_l1 += aj * jnp.dot(x, wl1_ref[j], preferred_element_type=jnp.float32)
    nsc_ref[...] = acc_sc * _INV_FAN
    nf_ref[...] = acc_l1 * _INV_FAN


def _node_prep(x, a, wsc_t, wl1_t):
    bn = 2000
    grid = _N // bn
    return pl.pallas_call(
        _node_prep_body,
        grid=(grid,),
        in_specs=[
            pl.BlockSpec((bn, _D), lambda i: (i, 0)),
            pl.BlockSpec((bn, _A), lambda i: (i, 0)),
            pl.BlockSpec((_A, _D, _D), lambda i: (0, 0, 0)),
            pl.BlockSpec((_A, _D, _D), lambda i: (0, 0, 0)),
        ],
        out_specs=[
            pl.BlockSpec((bn, _D), lambda i: (i, 0)),
            pl.BlockSpec((bn, _D), lambda i: (i, 0)),
        ],
        out_shape=[
            jax.ShapeDtypeStruct((_N, _D), jnp.float32),
            jax.ShapeDtypeStruct((_N, _D), jnp.float32),
        ],
    )(x, a, wsc_t, wl1_t)


def _edge_mlp_body(es_ref, attr_ref, wfc1_ref, wfc2_ref, out_ref):
    es = es_ref[...]
    h = jnp.dot(es, wfc1_ref[...], preferred_element_type=jnp.float32)
    h = h * (1.0 / math.sqrt(float(_S)))
    h = h * jax.nn.sigmoid(h)
    w = jnp.dot(h, wfc2_ref[...], preferred_element_type=jnp.float32)
    w = w * (1.0 / math.sqrt(float(_H)))
    out_ref[...] = w * attr_ref[...]


def _edge_mlp(es, attr_scaled, wfc1, wfc2):
    be = 4000
    grid = _E // be
    return pl.pallas_call(
        _edge_mlp_body,
        grid=(grid,),
        in_specs=[
            pl.BlockSpec((be, _S), lambda i: (i, 0)),
            pl.BlockSpec((be, 1), lambda i: (i, 0)),
            pl.BlockSpec((_S, _H), lambda i: (0, 0)),
            pl.BlockSpec((_H, _D), lambda i: (0, 0)),
        ],
        out_specs=pl.BlockSpec((be, _D), lambda i: (i, 0)),
        out_shape=jax.ShapeDtypeStruct((_E, _D), jnp.float32),
    )(es, attr_scaled, wfc1, wfc2)


def _edge_scatter_body(nf_hbm, w_hbm, src_hbm, dst_hbm, out_hbm,
                       sidx_v, didx_v, rows_v, wrow_v, acc_sh, sem):
    cid = lax.axis_index("c")
    sid = lax.axis_index("s")
    wid = sid * _NC + cid

    zero16 = jnp.zeros((16,), jnp.float32)

    def _zero_row(i, carry):
        for k in range(_D // 16):
            wrow_v[i, pl.ds(k * 16, 16)] = zero16
        return carry

    lax.fori_loop(0, _C, _zero_row, 0)
    for jj in range(_RPS // _C):
        pltpu.sync_copy(wrow_v, acc_sh.at[pl.ds(sid * _RPS + jj * _C, _C)])
    plsc.subcore_barrier()

    def _chunk(c, carry):
        pltpu.sync_copy(src_hbm.at[wid, c], sidx_v.at[0])
        pltpu.sync_copy(dst_hbm.at[wid, c], didx_v.at[0])
        pltpu.async_copy(nf_hbm.at[sidx_v.at[0]], rows_v, sem).wait()
        pltpu.sync_copy(w_hbm.at[wid, c], wrow_v)

        def _mul(e, carry2):
            for k in range(_D // 16):
                sl = pl.ds(k * 16, 16)
                rows_v[e, sl] = rows_v[e, sl] * wrow_v[e, sl]
            return carry2

        lax.fori_loop(0, _C, _mul, 0)
        pltpu.sync_copy(rows_v, acc_sh.at[didx_v.at[0]], add=True)
        return carry

    lax.fori_loop(0, _NCHUNK, _chunk, 0)
    plsc.subcore_barrier()

    for jj in range(_RPS // _C):
        base = sid * _RPS + jj * _C
        pltpu.sync_copy(acc_sh.at[pl.ds(base, _C)], wrow_v)
        pltpu.sync_copy(wrow_v, out_hbm.at[cid, pl.ds(base, _C)])


_edge_scatter = functools.partial(
    pl.kernel,
    out_type=jax.ShapeDtypeStruct((_NC, _NP, _D), jnp.float32),
    mesh=plsc.VectorSubcoreMesh(core_axis_name="c", subcore_axis_name="s"),
    scratch_types=[
        pltpu.VMEM((1, _C), jnp.int32),
        pltpu.VMEM((1, _C), jnp.int32),
        pltpu.VMEM((_C, _D), jnp.float32),
        pltpu.VMEM((_C, _D), jnp.float32),
        pltpu.VMEM_SHARED((_NP, _D), jnp.float32),
        pltpu.SemaphoreType.DMA,
    ],
)(_edge_scatter_body)


def _post_body(p0_ref, p1_ref, a_ref, wl2_ref, w3_ref, nsc_ref, out_ref):
    agg = p0_ref[...] + p1_ref[...]
    a = a_ref[...]
    acc = jnp.zeros(agg.shape, jnp.float32)
    for j in range(_A):
        acc += a[:, j:j + 1] * jnp.dot(agg, wl2_ref[j], preferred_element_type=jnp.float32)
    conv = acc * _INV_FAN
    t = jnp.dot(agg, w3_ref[...], preferred_element_type=jnp.float32)
    angle = (0.1 * _INV_FAN) * jnp.sum(t * a, axis=1, keepdims=True)
    out_ref[...] = jnp.cos(angle) * nsc_ref[...] + jnp.sin(angle) * conv


def _post(p0, p1, a, wl2_t, w3_r, nsc):
    bn = 2000
    grid = _N // bn
    return pl.pallas_call(
        _post_body,
        grid=(grid,),
        in_specs=[
            pl.BlockSpec((bn, _D), lambda i: (i, 0)),
            pl.BlockSpec((bn, _D), lambda i: (i, 0)),
            pl.BlockSpec((bn, _A), lambda i: (i, 0)),
            pl.BlockSpec((_A, _D, _D), lambda i: (0, 0, 0)),
            pl.BlockSpec((_D, _A), lambda i: (0, 0)),
            pl.BlockSpec((bn, _D), lambda i: (i, 0)),
        ],
        out_specs=pl.BlockSpec((bn, _D), lambda i: (i, 0)),
        out_shape=jax.ShapeDtypeStruct((_N, _D), jnp.float32),
    )(p0, p1, a, wl2_t, w3_r, nsc)


def kernel(node_input, node_attr, edge_src, edge_dst, edge_attr, edge_scalars,
           num_neighbors, W_sc, W_lin1, W_fc1, W_fc2, W_lin2, W_lin3):
    wsc_t = jnp.transpose(W_sc, (1, 0, 2))
    wl1_t = jnp.transpose(W_lin1, (1, 0, 2))
    wl2_t = jnp.transpose(W_lin2, (1, 0, 2))
    w3_r = W_lin3.reshape(_D, _A)

    inv_nb = 1.0 / jnp.sqrt(jnp.asarray(num_neighbors, jnp.float32))
    attr_scaled = edge_attr.astype(jnp.float32) * inv_nb

    nsc, nf = _node_prep(node_input, node_attr, wsc_t, wl1_t)
    w_scaled = _edge_mlp(edge_scalars, attr_scaled, W_fc1, W_fc2)

    src3 = edge_src.astype(jnp.int32).reshape(_NW, _NCHUNK, _C)
    dst3 = edge_dst.astype(jnp.int32).reshape(_NW, _NCHUNK, _C)
    w4 = w_scaled.reshape(_NW, _NCHUNK, _C, _D)

    partials = _edge_scatter(nf, w4, src3, dst3)

    return _post(partials[0, :_N], partials[1, :_N], node_attr, wl2_t, w3_r, nsc)


# speedup vs baseline: 2.2773x; 1.2667x over previous
import functools
import math

import jax
import jax.numpy as jnp
from jax import lax
from jax.experimental import pallas as pl
from jax.experimental.pallas import tpu as pltpu
from jax.experimental.pallas import tpu_sc as plsc

_N = 10000
_E = 320000
_D = 128
_A = 8
_S = 16
_H = 64

_NC = 2
_NS = 16
_NW = _NC * _NS
_EPW = _E // _NW
_C = 40
_NCHUNK = _EPW // _C
_KB = 50
_NB = _NCHUNK // _KB
_NPAIR = _NCHUNK // 2
_NP = 10240
_RPS = _NP // _NS

_INV_FAN = 1.0 / math.sqrt(float(_D * _A))


def _fctp_body(x_ref, a_ref, w_ref, o_ref):
    x = x_ref[...]
    a = a_ref[...]
    acc = jnp.zeros(x.shape, jnp.float32)
    for j in range(_A):
        acc += a[:, j:j + 1] * jnp.dot(x, w_ref[j], preferred_element_type=jnp.float32)
    o_ref[...] = acc * _INV_FAN


def _fctp(x, a, w_t):
    bn = 2000
    return pl.pallas_call(
        _fctp_body,
        grid=(_N // bn,),
        in_specs=[
            pl.BlockSpec((bn, _D), lambda i: (i, 0)),
            pl.BlockSpec((bn, _A), lambda i: (i, 0)),
            pl.BlockSpec((_A, _D, _D), lambda i: (0, 0, 0)),
        ],
        out_specs=pl.BlockSpec((bn, _D), lambda i: (i, 0)),
        out_shape=jax.ShapeDtypeStruct((_N, _D), jnp.float32),
    )(x, a, w_t)


def _edge_mlp_body(es_ref, attr_ref, wfc1_ref, wfc2_ref, out_ref):
    es = es_ref[...]
    h = jnp.dot(es, wfc1_ref[...], preferred_element_type=jnp.float32)
    h = h * (1.0 / math.sqrt(float(_S)))
    h = h * jax.nn.sigmoid(h)
    w = jnp.dot(h, wfc2_ref[...], preferred_element_type=jnp.float32)
    w = w * (1.0 / math.sqrt(float(_H)))
    out_ref[...] = w * attr_ref[...]


def _edge_mlp(es, attr_scaled, wfc1, wfc2):
    be = 4000
    return pl.pallas_call(
        _edge_mlp_body,
        grid=(_E // be,),
        in_specs=[
            pl.BlockSpec((be, _S), lambda i: (i, 0)),
            pl.BlockSpec((be, 1), lambda i: (i, 0)),
            pl.BlockSpec((_S, _H), lambda i: (0, 0)),
            pl.BlockSpec((_H, _D), lambda i: (0, 0)),
        ],
        out_specs=pl.BlockSpec((be, _D), lambda i: (i, 0)),
        out_shape=jax.ShapeDtypeStruct((_E, _D), jnp.float32),
    )(es, attr_scaled, wfc1, wfc2)


def _mul_rows(rows, wrow):
    def _mul(e, carry):
        for k in range(_D // 16):
            sl = pl.ds(k * 16, 16)
            rows[e, sl] = rows[e, sl] * wrow[e, sl]
        return carry

    lax.fori_loop(0, _C, _mul, 0)


def _edge_scatter_body(nf_hbm, w_hbm, src_hbm, dst_hbm, out_hbm,
                       sidx_v, didx_v, rows0, rows1, wrow0, wrow1, acc_sh,
                       gs0, gs1, ws0, ws1, ss0, ss1):
    cid = lax.axis_index("c")
    sid = lax.axis_index("s")
    wid = sid * _NC + cid
    sems = (gs0, gs1, ws0, ws1, ss0, ss1)

    zero16 = jnp.zeros((16,), jnp.float32)

    def _zero_row(i, carry):
        for k in range(_D // 16):
            wrow0[i, pl.ds(k * 16, 16)] = zero16
        return carry

    lax.fori_loop(0, _C, _zero_row, 0)
    zdescs = []
    for jj in range(_RPS // _C):
        zdescs.append(pltpu.async_copy(
            wrow0, acc_sh.at[pl.ds(sid * _RPS + jj * _C, _C)], sems[jj % 6]))
    for d in zdescs:
        d.wait()

    pltpu.sync_copy(src_hbm.at[wid, 0], sidx_v)
    pltpu.sync_copy(dst_hbm.at[wid, 0], didx_v)
    pltpu.async_copy(nf_hbm.at[sidx_v.at[0]], rows0, gs0)
    pltpu.async_copy(w_hbm.at[wid, 0], wrow0, ws0)
    pltpu.async_copy(nf_hbm.at[sidx_v.at[1]], rows1, gs1)
    pltpu.async_copy(w_hbm.at[wid, 1], wrow1, ws1)

    plsc.subcore_barrier()

    def _pair(i, carry):
        c0 = 2 * i
        j0 = lax.rem(c0, _KB)
        j1 = j0 + 1
        pltpu.make_async_copy(nf_hbm.at[sidx_v.at[0]], rows0, gs0).wait()
        pltpu.make_async_copy(w_hbm.at[wid, 0], wrow0, ws0).wait()
        _mul_rows(rows0, wrow0)
        pltpu.async_copy(rows0, acc_sh.at[didx_v.at[j0]], ss0, add=True)
        pltpu.make_async_copy(nf_hbm.at[sidx_v.at[1]], rows1, gs1).wait()
        pltpu.make_async_copy(w_hbm.at[wid, 0], wrow1, ws1).wait()
        _mul_rows(rows1, wrow1)
        pltpu.sync_copy(rows1, acc_sh.at[didx_v.at[j1]], add=True)
        pltpu.make_async_copy(rows0, acc_sh.at[didx_v.at[0]], ss0).wait()

        @pl.when(i < _NPAIR - 1)
        def _refill():
            nb = i + 1

            @pl.when(lax.rem(nb, _KB // 2) == 0)
            def _next_batch():
                b = lax.div(nb, _KB // 2)
                pltpu.sync_copy(src_hbm.at[wid, b], sidx_v)
                pltpu.sync_copy(dst_hbm.at[wid, b], didx_v)

            c0n = 2 * nb
            j0n = lax.rem(c0n, _KB)
            pltpu.async_copy(nf_hbm.at[sidx_v.at[j0n]], rows0, gs0)
            pltpu.async_copy(w_hbm.at[wid, c0n], wrow0, ws0)
            pltpu.async_copy(nf_hbm.at[sidx_v.at[j0n + 1]], rows1, gs1)
            pltpu.async_copy(w_hbm.at[wid, c0n + 1], wrow1, ws1)

        return carry

    lax.fori_loop(0, _NPAIR, _pair, 0)
    plsc.subcore_barrier()

    base = sid * _RPS
    pltpu.sync_copy(acc_sh.at[pl.ds(base, _RPS)], out_hbm.at[cid, pl.ds(base, _RPS)])


_edge_scatter = functools.partial(
    pl.kernel,
    out_type=jax.ShapeDtypeStruct((_NC, _NP, _D), jnp.float32),
    mesh=plsc.VectorSubcoreMesh(core_axis_name="c", subcore_axis_name="s"),
    scratch_types=[
        pltpu.VMEM((_KB, _C), jnp.int32),
        pltpu.VMEM((_KB, _C), jnp.int32),
        pltpu.VMEM((_C, _D), jnp.float32),
        pltpu.VMEM((_C, _D), jnp.float32),
        pltpu.VMEM((_C, _D), jnp.float32),
        pltpu.VMEM((_C, _D), jnp.float32),
        pltpu.VMEM_SHARED((_NP, _D), jnp.float32),
        pltpu.SemaphoreType.DMA,
        pltpu.SemaphoreType.DMA,
        pltpu.SemaphoreType.DMA,
        pltpu.SemaphoreType.DMA,
        pltpu.SemaphoreType.DMA,
        pltpu.SemaphoreType.DMA,
    ],
)(_edge_scatter_body)


def _post_body(p0_ref, p1_ref, a_ref, wl2_ref, w3_ref, nsc_ref, out_ref):
    agg = p0_ref[...] + p1_ref[...]
    a = a_ref[...]
    acc = jnp.zeros(agg.shape, jnp.float32)
    for j in range(_A):
        acc += a[:, j:j + 1] * jnp.dot(agg, wl2_ref[j], preferred_element_type=jnp.float32)
    conv = acc * _INV_FAN
    t = jnp.dot(agg, w3_ref[...], preferred_element_type=jnp.float32)
    angle = (0.1 * _INV_FAN) * jnp.sum(t * a, axis=1, keepdims=True)
    out_ref[...] = jnp.cos(angle) * nsc_ref[...] + jnp.sin(angle) * conv


def _post(p0, p1, a, wl2_t, w3_r, nsc):
    bn = 2000
    return pl.pallas_call(
        _post_body,
        grid=(_N // bn,),
        in_specs=[
            pl.BlockSpec((bn, _D), lambda i: (i, 0)),
            pl.BlockSpec((bn, _D), lambda i: (i, 0)),
            pl.BlockSpec((bn, _A), lambda i: (i, 0)),
            pl.BlockSpec((_A, _D, _D), lambda i: (0, 0, 0)),
            pl.BlockSpec((_D, _A), lambda i: (0, 0)),
            pl.BlockSpec((bn, _D), lambda i: (i, 0)),
        ],
        out_specs=pl.BlockSpec((bn, _D), lambda i: (i, 0)),
        out_shape=jax.ShapeDtypeStruct((_N, _D), jnp.float32),
    )(p0, p1, a, wl2_t, w3_r, nsc)


def kernel(node_input, node_attr, edge_src, edge_dst, edge_attr, edge_scalars,
           num_neighbors, W_sc, W_lin1, W_fc1, W_fc2, W_lin2, W_lin3):
    wsc_t = jnp.transpose(W_sc, (1, 0, 2))
    wl1_t = jnp.transpose(W_lin1, (1, 0, 2))
    wl2_t = jnp.transpose(W_lin2, (1, 0, 2))
    w3_r = W_lin3.reshape(_D, _A)

    inv_nb = 1.0 / jnp.sqrt(jnp.asarray(num_neighbors, jnp.float32))
    attr_scaled = edge_attr.astype(jnp.float32) * inv_nb

    nf = _fctp(node_input, node_attr, wl1_t)
    w_scaled = _edge_mlp(edge_scalars, attr_scaled, W_fc1, W_fc2)

    src4 = edge_src.astype(jnp.int32).reshape(_NW, _NB, _KB, _C)
    dst4 = edge_dst.astype(jnp.int32).reshape(_NW, _NB, _KB, _C)
    w4 = w_scaled.reshape(_NW, _NCHUNK, _C, _D)

    partials = _edge_scatter(nf, w4, src4, dst4)

    nsc = _fctp(node_input, node_attr, wsc_t)

    return _post(partials[0, :_N], partials[1, :_N], node_attr, wl2_t, w3_r, nsc)
